# Initial kernel scaffold; baseline (speedup 1.0000x reference)
import jax
import jax.numpy as jnp
from jax.experimental import pallas as pl

_BR = 512


def _rev_body(x_ref, o_ref):
    o_ref[...] = jnp.flip(x_ref[...], axis=1)


def kernel(x):
    B, N, D = x.shape
    nb = N // _BR
    return pl.pallas_call(
        _rev_body,
        grid=(B, nb),
        in_specs=[pl.BlockSpec((1, _BR, D), lambda b, i: (b, nb - 1 - i, 0))],
        out_specs=pl.BlockSpec((1, _BR, D), lambda b, i: (b, i, 0)),
        out_shape=jax.ShapeDtypeStruct(x.shape, x.dtype),
    )(x)


# speedup vs baseline: 2.5684x; 2.5684x over previous
import functools

import jax
import jax.numpy as jnp
from jax import lax
from jax.experimental import pallas as pl
from jax.experimental.pallas import tpu as pltpu
from jax.experimental.pallas import tpu_sc as plsc

_R = 16


def kernel(x):
    B, N, D = x.shape
    M = B * N
    xf = x.reshape(M, D)
    NW = 32
    rows_per_w = M // NW
    n_chunks = rows_per_w // _R
    mesh = plsc.VectorSubcoreMesh(core_axis_name="c", subcore_axis_name="s")

    @functools.partial(
        pl.kernel,
        mesh=mesh,
        out_type=jax.ShapeDtypeStruct((M, D), jnp.float32),
        scratch_types=[
            pltpu.VMEM((_R,), jnp.int32),
            pltpu.VMEM((_R, D), jnp.float32),
            pltpu.SemaphoreType.DMA,
        ],
    )
    def k(x_hbm, out_hbm, idx_v, rows_v, sem):
        wid = lax.axis_index("s") * 2 + lax.axis_index("c")
        base = wid * rows_per_w

        def body(t, _):
            obase = base + t * _R
            b = obase // N
            src0 = 2 * b * N + N - 1 - obase
            idx_v[...] = jnp.full((_R,), src0, jnp.int32) - lax.iota(jnp.int32, _R)
            pltpu.async_copy(x_hbm.at[idx_v], rows_v, sem).wait()
            pltpu.sync_copy(rows_v, out_hbm.at[pl.ds(obase, _R)])
            return 0

        lax.fori_loop(0, n_chunks, body, 0)

    return k(xf).reshape(B, N, D)


# speedup vs baseline: 3.0822x; 1.2000x over previous
import functools

import jax
import jax.numpy as jnp
from jax import lax
from jax.experimental import pallas as pl
from jax.experimental.pallas import tpu as pltpu
from jax.experimental.pallas import tpu_sc as plsc

_R = 16
_NBUF = 3


def kernel(x):
    B, N, D = x.shape
    M = B * N
    xf = x.reshape(M, D)
    NW = 32
    rows_per_w = M // NW
    n_chunks = rows_per_w // _R
    mesh = plsc.VectorSubcoreMesh(core_axis_name="c", subcore_axis_name="s")

    @functools.partial(
        pl.kernel,
        mesh=mesh,
        out_type=jax.ShapeDtypeStruct((M, D), jnp.float32),
        scratch_types=[
            pltpu.VMEM((_NBUF, _R), jnp.int32),
            pltpu.VMEM((_NBUF, _R, D), jnp.float32),
        ]
        + [pltpu.SemaphoreType.DMA] * (2 * _NBUF),
    )
    def k(x_hbm, out_hbm, idx_v, rows_v, *sems):
        gsems, wsems = sems[:_NBUF], sems[_NBUF:]
        wid = lax.axis_index("s") * 2 + lax.axis_index("c")
        base = wid * rows_per_w

        def start_gather(t, slot):
            obase = base + t * _R
            b = obase // N
            src0 = 2 * b * N + N - 1 - obase
            idx_v[slot, :] = jnp.full((_R,), src0, jnp.int32) - lax.iota(
                jnp.int32, _R
            )
            pltpu.async_copy(x_hbm.at[idx_v.at[slot]], rows_v.at[slot], gsems[slot])

        def wait_gather(slot):
            pltpu.make_async_copy(
                x_hbm.at[idx_v.at[slot]], rows_v.at[slot], gsems[slot]
            ).wait()

        def start_write(t, slot):
            pltpu.async_copy(
                rows_v.at[slot], out_hbm.at[pl.ds(base + t * _R, _R)], wsems[slot]
            )

        def wait_write(slot):
            pltpu.make_async_copy(
                rows_v.at[slot], out_hbm.at[pl.ds(base, _R)], wsems[slot]
            ).wait()

        for s in range(_NBUF - 1):
            start_gather(s, s)

        def main_body(step, _):
            for u in range(_NBUF):
                t = step * _NBUF + u
                slot = u

                @pl.when(t < n_chunks)
                def _():
                    wait_gather(slot)
                    start_write(t, slot)
                    t2 = t + _NBUF - 1
                    slot2 = (u + _NBUF - 1) % _NBUF

                    @pl.when(t2 < n_chunks)
                    def _():
                        @pl.when(t2 >= _NBUF)
                        def _():
                            wait_write(slot2)

                        start_gather(t2, slot2)

            return 0

        nsteps = (n_chunks + _NBUF - 1) // _NBUF
        lax.fori_loop(0, nsteps, main_body, 0)
        for s in range(_NBUF):
            if any(t % _NBUF == s for t in range(max(0, n_chunks - _NBUF), n_chunks)):
                wait_write(s)

    return k(xf).reshape(B, N, D)
